# Initial kernel scaffold; baseline (speedup 1.0000x reference)
import jax
import jax.numpy as jnp
from jax.experimental import pallas as pl


def kernel(x, position_ids):
    raise NotImplementedError("write your pallas kernel here")


# speedup vs baseline: 1.1555x; 1.1555x over previous
import functools

import jax
import jax.numpy as jnp
from jax import lax
from jax.experimental import pallas as pl
from jax.experimental.pallas import tpu as pltpu
from jax.experimental.pallas import tpu_sc as plsc

_DIM = 128
_MAX_POS = 32768
_BASE = 10000.0

_NC = 2
_NS = 16
_NW = _NC * _NS

_CHUNK = 128


def _build_gather(n_rows: int):
    assert n_rows % (_NW * _CHUNK) == 0
    n_chunks = n_rows // (_NW * _CHUNK)
    rows_per_w = n_chunks * _CHUNK

    mesh = plsc.VectorSubcoreMesh(
        core_axis_name="c", subcore_axis_name="s",
        num_cores=_NC, num_subcores=_NS,
    )

    @functools.partial(
        pl.kernel,
        mesh=mesh,
        out_type=(
            jax.ShapeDtypeStruct((n_rows, _DIM), jnp.float32),
            jax.ShapeDtypeStruct((n_rows, _DIM), jnp.float32),
        ),
        scratch_types=[
            pltpu.VMEM((n_chunks, _CHUNK), jnp.int32),
            pltpu.VMEM((2, _CHUNK, _DIM), jnp.float32),
            pltpu.VMEM((2, _CHUNK, _DIM), jnp.float32),
            pltpu.SemaphoreType.DMA,
        ],
    )
    def gather(cos_hbm, sin_hbm, idx_hbm, out_cos, out_sin,
               idx_v, cbuf, sbuf, gsem):
        wid = lax.axis_index("s") * _NC + lax.axis_index("c")
        base = wid * rows_per_w
        pltpu.sync_copy(idx_hbm.at[wid], idx_v)
        for j in range(n_chunks):
            slot = j % 2
            gc = pltpu.async_copy(cos_hbm.at[idx_v.at[j]], cbuf.at[slot], gsem)
            gs = pltpu.async_copy(sin_hbm.at[idx_v.at[j]], sbuf.at[slot], gsem)
            gc.wait()
            gs.wait()
            off = base + j * _CHUNK
            pltpu.sync_copy(cbuf.at[slot], out_cos.at[pl.ds(off, _CHUNK)])
            pltpu.sync_copy(sbuf.at[slot], out_sin.at[pl.ds(off, _CHUNK)])

    return gather


def kernel(x, position_ids):
    inv_freq = 1.0 / (_BASE ** (jnp.arange(0, _DIM, 2, dtype=jnp.float32) / _DIM))
    t = jnp.arange(_MAX_POS, dtype=jnp.float32)
    freqs = jnp.outer(t, inv_freq)
    emb = jnp.concatenate([freqs, freqs], axis=-1)
    cos_cached = jnp.cos(emb).astype(x.dtype)
    sin_cached = jnp.sin(emb).astype(x.dtype)

    b, s = position_ids.shape
    n_rows = b * s
    idx = position_ids.reshape(_NW, n_rows // (_NW * _CHUNK), _CHUNK)
    cos_flat, sin_flat = _build_gather(n_rows)(cos_cached, sin_cached, idx)
    return (cos_flat.reshape(b, s, _DIM), sin_flat.reshape(b, s, _DIM))


# speedup vs baseline: 2.1121x; 1.8278x over previous
import functools

import jax
import jax.numpy as jnp
import numpy as np
from jax import lax
from jax.experimental import pallas as pl
from jax.experimental.pallas import tpu as pltpu
from jax.experimental.pallas import tpu_sc as plsc

_DIM = 128
_MAX_POS = 32768
_BASE = 10000.0

_inv_freq = (
    1.0 / (_BASE ** (np.arange(0, _DIM, 2, dtype=np.float32) / np.float32(_DIM)))
).astype(np.float32)
_t = np.arange(_MAX_POS, dtype=np.float32)
_freqs = (_t[:, None] * _inv_freq[None, :]).astype(np.float32)
_emb = np.concatenate([_freqs, _freqs], axis=-1)
_COS_CACHE = np.cos(_emb).astype(np.float32)
_SIN_CACHE = np.sin(_emb).astype(np.float32)

_NC = 2
_NS = 16
_NW = _NC * _NS

_CHUNK = 128


def _build_gather(n_rows: int):
    assert n_rows % (_NW * _CHUNK) == 0
    n_chunks = n_rows // (_NW * _CHUNK)
    rows_per_w = n_chunks * _CHUNK

    mesh = plsc.VectorSubcoreMesh(
        core_axis_name="c", subcore_axis_name="s",
        num_cores=_NC, num_subcores=_NS,
    )

    @functools.partial(
        pl.kernel,
        mesh=mesh,
        out_type=(
            jax.ShapeDtypeStruct((n_rows, _DIM), jnp.float32),
            jax.ShapeDtypeStruct((n_rows, _DIM), jnp.float32),
        ),
        scratch_types=[
            pltpu.VMEM((n_chunks, _CHUNK), jnp.int32),
            pltpu.VMEM((2, _CHUNK, _DIM), jnp.float32),
            pltpu.VMEM((2, _CHUNK, _DIM), jnp.float32),
            pltpu.SemaphoreType.DMA,
        ],
    )
    def gather(cos_hbm, sin_hbm, idx_hbm, out_cos, out_sin,
               idx_v, cbuf, sbuf, gsem):
        wid = lax.axis_index("s") * _NC + lax.axis_index("c")
        base = wid * rows_per_w
        pltpu.sync_copy(idx_hbm.at[wid], idx_v)
        for j in range(n_chunks):
            slot = j % 2
            gc = pltpu.async_copy(cos_hbm.at[idx_v.at[j]], cbuf.at[slot], gsem)
            gs = pltpu.async_copy(sin_hbm.at[idx_v.at[j]], sbuf.at[slot], gsem)
            gc.wait()
            gs.wait()
            off = base + j * _CHUNK
            pltpu.sync_copy(cbuf.at[slot], out_cos.at[pl.ds(off, _CHUNK)])
            pltpu.sync_copy(sbuf.at[slot], out_sin.at[pl.ds(off, _CHUNK)])

    return gather


def kernel(x, position_ids):
    cos_cached = jnp.asarray(_COS_CACHE).astype(x.dtype)
    sin_cached = jnp.asarray(_SIN_CACHE).astype(x.dtype)

    b, s = position_ids.shape
    n_rows = b * s
    idx = position_ids.reshape(_NW, n_rows // (_NW * _CHUNK), _CHUNK)
    cos_flat, sin_flat = _build_gather(n_rows)(cos_cached, sin_cached, idx)
    return (cos_flat.reshape(b, s, _DIM), sin_flat.reshape(b, s, _DIM))


# speedup vs baseline: 2.3494x; 1.1124x over previous
import functools

import jax
import jax.numpy as jnp
import numpy as np
from jax import lax
from jax.experimental import pallas as pl
from jax.experimental.pallas import tpu as pltpu
from jax.experimental.pallas import tpu_sc as plsc

_DIM = 128
_HALF = _DIM // 2
_MAX_POS = 32768
_BASE = 10000.0

_inv_freq = (
    1.0 / (_BASE ** (np.arange(0, _DIM, 2, dtype=np.float32) / np.float32(_DIM)))
).astype(np.float32)
_t = np.arange(_MAX_POS, dtype=np.float32)
_freqs = (_t[:, None] * _inv_freq[None, :]).astype(np.float32)
_TABLE = np.concatenate(
    [np.cos(_freqs).astype(np.float32), np.sin(_freqs).astype(np.float32)],
    axis=1,
)

_NC = 2
_NS = 16
_NW = _NC * _NS

_CHUNK = 128


def _build_gather(n_rows: int):
    assert n_rows % (_NW * _CHUNK) == 0
    n_chunks = n_rows // (_NW * _CHUNK)
    rows_per_w = n_chunks * _CHUNK

    mesh = plsc.VectorSubcoreMesh(
        core_axis_name="c", subcore_axis_name="s",
        num_cores=_NC, num_subcores=_NS,
    )

    @functools.partial(
        pl.kernel,
        mesh=mesh,
        out_type=jax.ShapeDtypeStruct((n_rows, _DIM), jnp.float32),
        scratch_types=[
            pltpu.VMEM((n_chunks, _CHUNK), jnp.int32),
            pltpu.VMEM((n_chunks, _CHUNK, _DIM), jnp.float32),
            pltpu.SemaphoreType.DMA((n_chunks,)),
            pltpu.SemaphoreType.DMA((n_chunks,)),
        ],
    )
    def gather(tbl_hbm, idx_hbm, out, idx_v, buf, gsem, ssem):
        wid = lax.axis_index("s") * _NC + lax.axis_index("c")
        base = wid * rows_per_w
        pltpu.sync_copy(idx_hbm.at[wid], idx_v)
        gets = [
            pltpu.async_copy(tbl_hbm.at[idx_v.at[j]], buf.at[j], gsem.at[j])
            for j in range(n_chunks)
        ]
        puts = []
        for j in range(n_chunks):
            gets[j].wait()
            rows = pl.ds(base + j * _CHUNK, _CHUNK)
            puts.append(pltpu.async_copy(buf.at[j], out.at[rows], ssem.at[j]))
        for p in puts:
            p.wait()

    return gather


def _expand_body(comb_ref, cos_ref, sin_ref):
    v = comb_ref[...]
    ch = v[:, :_HALF]
    sh = v[:, _HALF:]
    cos_ref[...] = jnp.concatenate([ch, ch], axis=1)
    sin_ref[...] = jnp.concatenate([sh, sh], axis=1)


def _expand(comb):
    n_rows = comb.shape[0]
    blk = 2048
    out = jax.ShapeDtypeStruct((n_rows, _DIM), jnp.float32)
    return pl.pallas_call(
        _expand_body,
        grid=(n_rows // blk,),
        in_specs=[pl.BlockSpec((blk, _DIM), lambda i: (i, 0))],
        out_specs=[pl.BlockSpec((blk, _DIM), lambda i: (i, 0))] * 2,
        out_shape=[out, out],
    )(comb)


def kernel(x, position_ids):
    tbl = jnp.asarray(_TABLE).astype(x.dtype)

    b, s = position_ids.shape
    n_rows = b * s
    idx = position_ids.reshape(_NW, n_rows // (_NW * _CHUNK), _CHUNK)
    comb = _build_gather(n_rows)(tbl, idx)
    cos, sin = _expand(comb)
    return (cos.reshape(b, s, _DIM), sin.reshape(b, s, _DIM))
